# Initial kernel scaffold; baseline (speedup 1.0000x reference)
import jax
import jax.numpy as jnp
from jax.experimental import pallas as pl


def kernel(x, tables):
    raise NotImplementedError("write your pallas kernel here")


# speedup vs baseline: 1.1176x; 1.1176x over previous
import functools

import jax
import jax.numpy as jnp
from jax import lax
from jax.experimental import pallas as pl
from jax.experimental.pallas import tpu as pltpu
from jax.experimental.pallas import tpu_sc as plsc

NUM_FEATURES = 26
VOCAB = 100000
EMBED = 32
BATCH = 16384

NC = 2
NS = 16
NW = NC * NS

TOTAL_ROWS = BATCH * NUM_FEATURES
ROWS_W = TOTAL_ROWS // NW
IDX_COLS = 128
IDX_ROWS = ROWS_W // IDX_COLS
GATHERS_PER_CHUNK = 4
CHUNK = GATHERS_PER_CHUNK * IDX_COLS
N_CHUNKS = ROWS_W // CHUNK
LANES = 16
VECS_PER_IDX_ROW = IDX_COLS // LANES


def _embed_body(x3, tab, out, idx_v, rows_v, gsem):
    wid = lax.axis_index("s") * NC + lax.axis_index("c")
    base = wid * ROWS_W

    pltpu.sync_copy(x3.at[wid], idx_v)

    def add_offsets(j, carry):
        row0 = j * IDX_COLS
        for k in range(VECS_PER_IDX_ROW):
            pos = lax.iota(jnp.int32, LANES) + (row0 + k * LANES)
            feat = lax.rem(pos, jnp.int32(NUM_FEATURES))
            sl = pl.ds(k * LANES, LANES)
            idx_v[j, sl] = idx_v[j, sl] + feat * jnp.int32(VOCAB)
        return carry

    lax.fori_loop(0, IDX_ROWS, add_offsets, 0)

    def chunk_body(c, carry):
        copies = []
        for k in range(GATHERS_PER_CHUNK):
            cp = pltpu.async_copy(
                tab.at[idx_v.at[c * GATHERS_PER_CHUNK + k]],
                rows_v.at[pl.ds(k * IDX_COLS, IDX_COLS)],
                gsem,
            )
            copies.append(cp)
        for cp in copies:
            cp.wait()
        pltpu.sync_copy(rows_v, out.at[pl.ds(base + c * CHUNK, CHUNK)])
        return carry

    lax.fori_loop(0, N_CHUNKS, chunk_body, 0)


def kernel(x, tables):
    x3 = x.astype(jnp.int32).reshape(NW, IDX_ROWS, IDX_COLS)
    tab = tables.reshape(NUM_FEATURES * VOCAB, EMBED)

    run = functools.partial(
        pl.kernel,
        out_type=jax.ShapeDtypeStruct((TOTAL_ROWS, EMBED), jnp.float32),
        mesh=plsc.VectorSubcoreMesh(core_axis_name="c", subcore_axis_name="s"),
        compiler_params=pltpu.CompilerParams(use_tc_tiling_on_sc=False),
        scratch_types=[
            pltpu.VMEM((IDX_ROWS, IDX_COLS), jnp.int32),
            pltpu.VMEM((CHUNK, EMBED), jnp.float32),
            pltpu.SemaphoreType.DMA,
        ],
    )(_embed_body)

    out_flat = run(x3, tab)
    return out_flat.reshape(BATCH, NUM_FEATURES, EMBED)


# speedup vs baseline: 1.1437x; 1.0233x over previous
import functools

import jax
import jax.numpy as jnp
from jax import lax
from jax.experimental import pallas as pl
from jax.experimental.pallas import tpu as pltpu
from jax.experimental.pallas import tpu_sc as plsc

NUM_FEATURES = 26
VOCAB = 100000
EMBED = 32
BATCH = 16384

NC = 2
NS = 16
NW = NC * NS

B_W = BATCH // NW
IDX_SLICE = 128
GATHERS = B_W // IDX_SLICE


def _embed_body(xt, tab, out, idx_v, rows_v, gsem):
    wid = lax.axis_index("s") * NC + lax.axis_index("c")
    b0 = wid * B_W

    def feature_body(f, carry):
        pltpu.sync_copy(xt.at[f, pl.ds(b0, B_W)], idx_v)
        copies = []
        for k in range(GATHERS):
            cp = pltpu.async_copy(
                tab.at[f].at[idx_v.at[pl.ds(k * IDX_SLICE, IDX_SLICE)]],
                rows_v.at[pl.ds(k * IDX_SLICE, IDX_SLICE)],
                gsem,
            )
            copies.append(cp)
        for cp in copies:
            cp.wait()
        pltpu.sync_copy(rows_v, out.at[f, pl.ds(b0, B_W)])
        return carry

    lax.fori_loop(0, NUM_FEATURES, feature_body, 0)


def kernel(x, tables):
    xt = x.T

    run = functools.partial(
        pl.kernel,
        out_type=jax.ShapeDtypeStruct((NUM_FEATURES, BATCH, EMBED), jnp.float32),
        mesh=plsc.VectorSubcoreMesh(core_axis_name="c", subcore_axis_name="s"),
        compiler_params=pltpu.CompilerParams(use_tc_tiling_on_sc=False),
        scratch_types=[
            pltpu.VMEM((B_W,), jnp.int32),
            pltpu.VMEM((B_W, EMBED), jnp.float32),
            pltpu.SemaphoreType.DMA,
        ],
    )(_embed_body)

    out_t = run(xt, tables)
    return out_t.transpose(1, 0, 2)
